# baseline (speedup 1.0000x reference)
import functools

import numpy as np

import jax
import jax.numpy as jnp
from jax import lax
from jax.experimental import pallas as pl
from jax.experimental.pallas import tpu as pltpu
from jax.experimental.pallas import tpu_sc as plsc

_LANES = 16
_NC = 2
_NS = 16
_NW = _NC * _NS
_C = 80
_NB = 2


def _make_sc_call(n_triplets: int, feat: int):
    assert feat % (2 * _LANES) == 0
    fw = feat // 2
    per_w = n_triplets // _NW
    assert per_w * _NW == n_triplets
    assert per_w % _C == 0 and _C % _LANES == 0
    n_chunks = per_w // _C
    d_chunks = fw // _LANES

    mesh = plsc.VectorSubcoreMesh(core_axis_name="c", subcore_axis_name="s")

    def body(h_hbm, s_hbm, r_hbm, o_hbm, w_hbm, out_hbm,
             s_idx, r_idx, o_idx, hs, wr, ho, tile, out_v, semg):
        wid = lax.axis_index("s") * _NC + lax.axis_index("c")
        base = wid * per_w

        pltpu.sync_copy(s_hbm.at[pl.ds(base, per_w)], s_idx)
        pltpu.sync_copy(r_hbm.at[pl.ds(base, per_w)], r_idx)
        pltpu.sync_copy(o_hbm.at[pl.ds(base, per_w)], o_idx)

        lane = lax.iota(jnp.int32, _LANES)

        def issue(c, b):
            off = c * _C
            pltpu.async_copy(h_hbm.at[s_idx.at[pl.ds(off, _C)]],
                             hs.at[b], semg.at[b])
            pltpu.async_copy(w_hbm.at[r_idx.at[pl.ds(off, _C)]],
                             wr.at[b], semg.at[b])
            pltpu.async_copy(h_hbm.at[o_idx.at[pl.ds(off, _C)]],
                             ho.at[b], semg.at[b])

        def wait3(b):
            pltpu.make_async_copy(h_hbm.at[s_idx.at[pl.ds(0, _C)]],
                                  hs.at[b], semg.at[b]).wait()
            pltpu.make_async_copy(w_hbm.at[r_idx.at[pl.ds(0, _C)]],
                                  wr.at[b], semg.at[b]).wait()
            pltpu.make_async_copy(h_hbm.at[o_idx.at[pl.ds(0, _C)]],
                                  ho.at[b], semg.at[b]).wait()

        def compute(c, b):
            off = c * _C
            for g in range(_C // _LANES):
                for j in range(_LANES):
                    row = g * _LANES + j
                    acc0 = jnp.zeros((_LANES,), jnp.float32)
                    acc1 = jnp.zeros((_LANES,), jnp.float32)
                    hi_mask = jnp.full((_LANES,), -65536, jnp.int32)
                    for d in range(d_chunks):
                        sl = pl.ds(d * _LANES, _LANES)
                        aw = hs[b, row, sl]
                        bw = wr[b, row, sl]
                        cw = ho[b, row, sl]
                        a0 = plsc.bitcast(aw << 16, jnp.float32)
                        a1 = plsc.bitcast(aw & hi_mask, jnp.float32)
                        b0 = plsc.bitcast(bw << 16, jnp.float32)
                        b1 = plsc.bitcast(bw & hi_mask, jnp.float32)
                        c0 = plsc.bitcast(cw << 16, jnp.float32)
                        c1 = plsc.bitcast(cw & hi_mask, jnp.float32)
                        acc0 = acc0 + a0 * b0 * c0
                        acc1 = acc1 + a1 * b1 * c1
                    tile[j, :] = acc0 + acc1
                parts = [jnp.zeros((_LANES,), jnp.float32) for _ in range(4)]
                for d in range(_LANES):
                    col = jnp.full((_LANES,), d, jnp.int32)
                    parts[d % 4] = parts[d % 4] + plsc.load_gather(
                        tile, [lane, col])
                scores = (parts[0] + parts[1]) + (parts[2] + parts[3])
                out_v[pl.ds(off + g * _LANES, _LANES)] = scores

        for c in range(_NB - 1):
            issue(c, c)

        def step(c, carry):
            b = c & (_NB - 1)
            issue(c + _NB - 1, (c + _NB - 1) & (_NB - 1))
            wait3(b)
            compute(c, b)
            return carry

        lax.fori_loop(0, n_chunks - _NB + 1, step, 0)
        for c in range(n_chunks - _NB + 1, n_chunks):
            b = c & (_NB - 1)
            wait3(b)
            compute(c, b)

        pltpu.sync_copy(out_v, out_hbm.at[pl.ds(base, per_w)])

    return functools.partial(
        pl.kernel,
        out_type=jax.ShapeDtypeStruct((n_triplets,), jnp.float32),
        mesh=mesh,
        compiler_params=pltpu.CompilerParams(
            needs_layout_passes=False, use_tc_tiling_on_sc=False),
        scratch_types=[
            pltpu.VMEM((per_w,), jnp.int32),
            pltpu.VMEM((per_w,), jnp.int32),
            pltpu.VMEM((per_w,), jnp.int32),
            pltpu.VMEM((_NB, _C, fw), jnp.int32),
            pltpu.VMEM((_NB, _C, fw), jnp.int32),
            pltpu.VMEM((_NB, _C, fw), jnp.int32),
            pltpu.VMEM((_LANES, _LANES), jnp.float32),
            pltpu.VMEM((per_w,), jnp.float32),
            pltpu.SemaphoreType.DMA((_NB,)),
        ],
    )(body)


def kernel(h, triplets, W):
    n_triplets = triplets.shape[0]
    feat = h.shape[1]
    call = _make_sc_call(n_triplets, feat)
    s = triplets[:, 0]
    r = triplets[:, 1]
    o = triplets[:, 2]

    def pack32(x):
        x16 = x.astype(jnp.bfloat16)
        return lax.bitcast_convert_type(
            x16.reshape(x.shape[0], x.shape[1] // 2, 2), jnp.int32)

    return call(pack32(h), s, r, o, pack32(W))


# speedup vs baseline: 1.6921x; 1.6921x over previous
import functools

import numpy as np

import jax
import jax.numpy as jnp
from jax import lax
from jax.experimental import pallas as pl
from jax.experimental.pallas import tpu as pltpu
from jax.experimental.pallas import tpu_sc as plsc

_LANES = 16
_NC = 2
_NS = 16
_NW = _NC * _NS
_C = 80
_NB = 2


def _make_sc_call(n_triplets: int, feat: int):
    assert feat % (2 * _LANES) == 0
    fw = feat // 2
    per_w = n_triplets // _NW
    assert per_w * _NW == n_triplets
    assert per_w % _C == 0 and _C % _LANES == 0
    n_chunks = per_w // _C
    d_chunks = fw // _LANES

    mesh = plsc.VectorSubcoreMesh(core_axis_name="c", subcore_axis_name="s")

    def body(h_hbm, s_hbm, r_hbm, o_hbm, w_hbm, out_hbm,
             s_idx, r_idx, o_idx, hs, wr, ho, tile, out_v, semg):
        wid = lax.axis_index("s") * _NC + lax.axis_index("c")
        base = wid * per_w

        pltpu.sync_copy(s_hbm.at[pl.ds(base, per_w)], s_idx)
        pltpu.sync_copy(r_hbm.at[pl.ds(base, per_w)], r_idx)
        pltpu.sync_copy(o_hbm.at[pl.ds(base, per_w)], o_idx)

        lane = lax.iota(jnp.int32, _LANES)

        def issue(c, b):
            off = c * _C
            pltpu.async_copy(h_hbm.at[s_idx.at[pl.ds(off, _C)]],
                             hs.at[b], semg.at[b])
            pltpu.async_copy(w_hbm.at[r_idx.at[pl.ds(off, _C)]],
                             wr.at[b], semg.at[b])
            pltpu.async_copy(h_hbm.at[o_idx.at[pl.ds(off, _C)]],
                             ho.at[b], semg.at[b])

        def wait3(b):
            pltpu.make_async_copy(h_hbm.at[s_idx.at[pl.ds(0, _C)]],
                                  hs.at[b], semg.at[b]).wait()
            pltpu.make_async_copy(w_hbm.at[r_idx.at[pl.ds(0, _C)]],
                                  wr.at[b], semg.at[b]).wait()
            pltpu.make_async_copy(h_hbm.at[o_idx.at[pl.ds(0, _C)]],
                                  ho.at[b], semg.at[b]).wait()

        def compute(c, b):
            off = c * _C

            def group(g, carry):
                for j in range(_LANES):
                    row = g * _LANES + j
                    acc0 = jnp.zeros((_LANES,), jnp.float32)
                    acc1 = jnp.zeros((_LANES,), jnp.float32)
                    hi_mask = jnp.full((_LANES,), -65536, jnp.int32)
                    for d in range(d_chunks):
                        sl = pl.ds(d * _LANES, _LANES)
                        aw = hs[b, row, sl]
                        bw = wr[b, row, sl]
                        cw = ho[b, row, sl]
                        a0 = plsc.bitcast(aw << 16, jnp.float32)
                        a1 = plsc.bitcast(aw & hi_mask, jnp.float32)
                        b0 = plsc.bitcast(bw << 16, jnp.float32)
                        b1 = plsc.bitcast(bw & hi_mask, jnp.float32)
                        c0 = plsc.bitcast(cw << 16, jnp.float32)
                        c1 = plsc.bitcast(cw & hi_mask, jnp.float32)
                        acc0 = acc0 + a0 * b0 * c0
                        acc1 = acc1 + a1 * b1 * c1
                    tile[j, :] = acc0 + acc1
                parts = [jnp.zeros((_LANES,), jnp.float32) for _ in range(4)]
                for d in range(_LANES):
                    col = jnp.full((_LANES,), d, jnp.int32)
                    parts[d % 4] = parts[d % 4] + plsc.load_gather(
                        tile, [lane, col])
                scores = (parts[0] + parts[1]) + (parts[2] + parts[3])
                out_v[pl.ds(off + g * _LANES, _LANES)] = scores
                return carry

            lax.fori_loop(0, _C // _LANES, group, 0)

        for c in range(_NB - 1):
            issue(c, c)

        def step(c, carry):
            b = c & (_NB - 1)
            issue(c + _NB - 1, (c + _NB - 1) & (_NB - 1))
            wait3(b)
            compute(c, b)
            return carry

        lax.fori_loop(0, n_chunks - _NB + 1, step, 0)
        for c in range(n_chunks - _NB + 1, n_chunks):
            b = c & (_NB - 1)
            wait3(b)
            compute(c, b)

        pltpu.sync_copy(out_v, out_hbm.at[pl.ds(base, per_w)])

    return functools.partial(
        pl.kernel,
        out_type=jax.ShapeDtypeStruct((n_triplets,), jnp.float32),
        mesh=mesh,
        compiler_params=pltpu.CompilerParams(
            needs_layout_passes=False, use_tc_tiling_on_sc=False),
        scratch_types=[
            pltpu.VMEM((per_w,), jnp.int32),
            pltpu.VMEM((per_w,), jnp.int32),
            pltpu.VMEM((per_w,), jnp.int32),
            pltpu.VMEM((_NB, _C, fw), jnp.int32),
            pltpu.VMEM((_NB, _C, fw), jnp.int32),
            pltpu.VMEM((_NB, _C, fw), jnp.int32),
            pltpu.VMEM((_LANES, _LANES), jnp.float32),
            pltpu.VMEM((per_w,), jnp.float32),
            pltpu.SemaphoreType.DMA((_NB,)),
        ],
    )(body)


def kernel(h, triplets, W):
    n_triplets = triplets.shape[0]
    feat = h.shape[1]
    call = _make_sc_call(n_triplets, feat)
    s = triplets[:, 0]
    r = triplets[:, 1]
    o = triplets[:, 2]

    def pack32(x):
        x16 = x.astype(jnp.bfloat16)
        return lax.bitcast_convert_type(
            x16.reshape(x.shape[0], x.shape[1] // 2, 2), jnp.int32)

    return call(pack32(h), s, r, o, pack32(W))


# speedup vs baseline: 2.1294x; 1.2584x over previous
import functools

import numpy as np

import jax
import jax.numpy as jnp
from jax import lax
from jax.experimental import pallas as pl
from jax.experimental.pallas import tpu as pltpu
from jax.experimental.pallas import tpu_sc as plsc

_LANES = 16
_NC = 2
_NS = 16
_NW = _NC * _NS
_C = 80
_NB = 2


def _make_sc_call(n_triplets: int, feat: int):
    assert feat % (2 * _LANES) == 0
    fw = feat // 2
    per_w = n_triplets // _NW
    assert per_w * _NW == n_triplets
    assert per_w % _C == 0 and _C % _LANES == 0
    n_chunks = per_w // _C
    d_chunks = fw // _LANES

    mesh = plsc.VectorSubcoreMesh(core_axis_name="c", subcore_axis_name="s")

    def body(h_hbm, s_hbm, r_hbm, o_hbm, w_hbm, out_hbm,
             s_idx, r_idx, o_idx, hs, wr, ho, tile, out_v, semg):
        wid = lax.axis_index("s") * _NC + lax.axis_index("c")
        base = wid * per_w

        pltpu.sync_copy(s_hbm.at[pl.ds(base, per_w)], s_idx)
        pltpu.sync_copy(r_hbm.at[pl.ds(base, per_w)], r_idx)
        pltpu.sync_copy(o_hbm.at[pl.ds(base, per_w)], o_idx)

        lane = lax.iota(jnp.int32, _LANES)

        def issue(c, b):
            off = c * _C
            pltpu.async_copy(h_hbm.at[s_idx.at[pl.ds(off, _C)]],
                             hs.at[b], semg.at[b])
            pltpu.async_copy(w_hbm.at[r_idx.at[pl.ds(off, _C)]],
                             wr.at[b], semg.at[b])
            pltpu.async_copy(h_hbm.at[o_idx.at[pl.ds(off, _C)]],
                             ho.at[b], semg.at[b])

        def wait3(b):
            pltpu.make_async_copy(h_hbm.at[s_idx.at[pl.ds(0, _C)]],
                                  hs.at[b], semg.at[b]).wait()
            pltpu.make_async_copy(w_hbm.at[r_idx.at[pl.ds(0, _C)]],
                                  wr.at[b], semg.at[b]).wait()
            pltpu.make_async_copy(h_hbm.at[o_idx.at[pl.ds(0, _C)]],
                                  ho.at[b], semg.at[b]).wait()

        def compute(c, b):
            off = c * _C

            def group(g, carry):
                scores = jnp.zeros((_LANES,), jnp.float32)
                for j in range(_LANES):
                    row = g * _LANES + j
                    acc0 = jnp.zeros((_LANES,), jnp.float32)
                    acc1 = jnp.zeros((_LANES,), jnp.float32)
                    hi_mask = jnp.full((_LANES,), -65536, jnp.int32)
                    for d in range(d_chunks):
                        sl = pl.ds(d * _LANES, _LANES)
                        aw = hs[b, row, sl]
                        bw = wr[b, row, sl]
                        cw = ho[b, row, sl]
                        a0 = plsc.bitcast(aw << 16, jnp.float32)
                        a1 = plsc.bitcast(aw & hi_mask, jnp.float32)
                        b0 = plsc.bitcast(bw << 16, jnp.float32)
                        b1 = plsc.bitcast(bw & hi_mask, jnp.float32)
                        c0 = plsc.bitcast(cw << 16, jnp.float32)
                        c1 = plsc.bitcast(cw & hi_mask, jnp.float32)
                        acc0 = acc0 + a0 * b0 * c0
                        acc1 = acc1 + a1 * b1 * c1
                    tot = jnp.sum(acc0 + acc1)
                    scores = jnp.where(lane == j, tot, scores)
                out_v[pl.ds(off + g * _LANES, _LANES)] = scores
                return carry

            lax.fori_loop(0, _C // _LANES, group, 0)

        for c in range(_NB - 1):
            issue(c, c)

        def step(c, carry):
            b = c & (_NB - 1)
            issue(c + _NB - 1, (c + _NB - 1) & (_NB - 1))
            wait3(b)
            compute(c, b)
            return carry

        lax.fori_loop(0, n_chunks - _NB + 1, step, 0)
        for c in range(n_chunks - _NB + 1, n_chunks):
            b = c & (_NB - 1)
            wait3(b)
            compute(c, b)

        pltpu.sync_copy(out_v, out_hbm.at[pl.ds(base, per_w)])

    return functools.partial(
        pl.kernel,
        out_type=jax.ShapeDtypeStruct((n_triplets,), jnp.float32),
        mesh=mesh,
        compiler_params=pltpu.CompilerParams(
            needs_layout_passes=False, use_tc_tiling_on_sc=False),
        scratch_types=[
            pltpu.VMEM((per_w,), jnp.int32),
            pltpu.VMEM((per_w,), jnp.int32),
            pltpu.VMEM((per_w,), jnp.int32),
            pltpu.VMEM((_NB, _C, fw), jnp.int32),
            pltpu.VMEM((_NB, _C, fw), jnp.int32),
            pltpu.VMEM((_NB, _C, fw), jnp.int32),
            pltpu.VMEM((_LANES, _LANES), jnp.float32),
            pltpu.VMEM((per_w,), jnp.float32),
            pltpu.SemaphoreType.DMA((_NB,)),
        ],
    )(body)


def kernel(h, triplets, W):
    n_triplets = triplets.shape[0]
    feat = h.shape[1]
    call = _make_sc_call(n_triplets, feat)
    s = triplets[:, 0]
    r = triplets[:, 1]
    o = triplets[:, 2]

    def pack32(x):
        x16 = x.astype(jnp.bfloat16)
        return lax.bitcast_convert_type(
            x16.reshape(x.shape[0], x.shape[1] // 2, 2), jnp.int32)

    return call(pack32(h), s, r, o, pack32(W))


# speedup vs baseline: 2.3674x; 1.1118x over previous
import functools

import numpy as np

import jax
import jax.numpy as jnp
from jax import lax
from jax.experimental import pallas as pl
from jax.experimental.pallas import tpu as pltpu
from jax.experimental.pallas import tpu_sc as plsc

_LANES = 16
_NC = 2
_NS = 16
_NW = _NC * _NS
_C = 80
_NB = 4


def _make_sc_call(n_triplets: int, feat: int):
    assert feat % (2 * _LANES) == 0
    fw = feat // 2
    per_w = n_triplets // _NW
    assert per_w * _NW == n_triplets
    assert per_w % _C == 0 and _C % _LANES == 0
    n_chunks = per_w // _C
    d_chunks = fw // _LANES

    mesh = plsc.VectorSubcoreMesh(core_axis_name="c", subcore_axis_name="s")

    def body(h_hbm, s_hbm, r_hbm, o_hbm, w_hbm, out_hbm,
             s_idx, r_idx, o_idx, hs, wr, ho, tile, out_v, semg):
        wid = lax.axis_index("s") * _NC + lax.axis_index("c")
        base = wid * per_w

        pltpu.sync_copy(s_hbm.at[pl.ds(base, per_w)], s_idx)
        pltpu.sync_copy(r_hbm.at[pl.ds(base, per_w)], r_idx)
        pltpu.sync_copy(o_hbm.at[pl.ds(base, per_w)], o_idx)

        lane = lax.iota(jnp.int32, _LANES)

        def issue(c, b):
            off = c * _C
            pltpu.async_copy(h_hbm.at[s_idx.at[pl.ds(off, _C)]],
                             hs.at[b], semg.at[b])
            pltpu.async_copy(w_hbm.at[r_idx.at[pl.ds(off, _C)]],
                             wr.at[b], semg.at[b])
            pltpu.async_copy(h_hbm.at[o_idx.at[pl.ds(off, _C)]],
                             ho.at[b], semg.at[b])

        def wait3(b):
            pltpu.make_async_copy(h_hbm.at[s_idx.at[pl.ds(0, _C)]],
                                  hs.at[b], semg.at[b]).wait()
            pltpu.make_async_copy(w_hbm.at[r_idx.at[pl.ds(0, _C)]],
                                  wr.at[b], semg.at[b]).wait()
            pltpu.make_async_copy(h_hbm.at[o_idx.at[pl.ds(0, _C)]],
                                  ho.at[b], semg.at[b]).wait()

        def compute(c, b):
            off = c * _C

            def group(g, carry):
                scores = jnp.zeros((_LANES,), jnp.float32)
                for j in range(_LANES):
                    row = g * _LANES + j
                    acc0 = jnp.zeros((_LANES,), jnp.float32)
                    acc1 = jnp.zeros((_LANES,), jnp.float32)
                    hi_mask = jnp.full((_LANES,), -65536, jnp.int32)
                    for d in range(d_chunks):
                        sl = pl.ds(d * _LANES, _LANES)
                        aw = hs[b, row, sl]
                        bw = wr[b, row, sl]
                        cw = ho[b, row, sl]
                        a0 = plsc.bitcast(aw << 16, jnp.float32)
                        a1 = plsc.bitcast(aw & hi_mask, jnp.float32)
                        b0 = plsc.bitcast(bw << 16, jnp.float32)
                        b1 = plsc.bitcast(bw & hi_mask, jnp.float32)
                        c0 = plsc.bitcast(cw << 16, jnp.float32)
                        c1 = plsc.bitcast(cw & hi_mask, jnp.float32)
                        acc0 = acc0 + a0 * b0 * c0
                        acc1 = acc1 + a1 * b1 * c1
                    tot = jnp.sum(acc0 + acc1)
                    scores = jnp.where(lane == j, tot, scores)
                out_v[pl.ds(off + g * _LANES, _LANES)] = scores
                return carry

            lax.fori_loop(0, _C // _LANES, group, 0)

        for c in range(_NB - 1):
            issue(c, c)

        def step(c, carry):
            b = c & (_NB - 1)
            issue(c + _NB - 1, (c + _NB - 1) & (_NB - 1))
            wait3(b)
            compute(c, b)
            return carry

        lax.fori_loop(0, n_chunks - _NB + 1, step, 0)
        for c in range(n_chunks - _NB + 1, n_chunks):
            b = c & (_NB - 1)
            wait3(b)
            compute(c, b)

        pltpu.sync_copy(out_v, out_hbm.at[pl.ds(base, per_w)])

    return functools.partial(
        pl.kernel,
        out_type=jax.ShapeDtypeStruct((n_triplets,), jnp.float32),
        mesh=mesh,
        compiler_params=pltpu.CompilerParams(
            needs_layout_passes=False, use_tc_tiling_on_sc=False),
        scratch_types=[
            pltpu.VMEM((per_w,), jnp.int32),
            pltpu.VMEM((per_w,), jnp.int32),
            pltpu.VMEM((per_w,), jnp.int32),
            pltpu.VMEM((_NB, _C, fw), jnp.int32),
            pltpu.VMEM((_NB, _C, fw), jnp.int32),
            pltpu.VMEM((_NB, _C, fw), jnp.int32),
            pltpu.VMEM((_LANES, _LANES), jnp.float32),
            pltpu.VMEM((per_w,), jnp.float32),
            pltpu.SemaphoreType.DMA((_NB,)),
        ],
    )(body)


def kernel(h, triplets, W):
    n_triplets = triplets.shape[0]
    feat = h.shape[1]
    call = _make_sc_call(n_triplets, feat)
    s = triplets[:, 0]
    r = triplets[:, 1]
    o = triplets[:, 2]

    def pack32(x):
        x16 = x.astype(jnp.bfloat16)
        return lax.bitcast_convert_type(
            x16.reshape(x.shape[0], x.shape[1] // 2, 2), jnp.int32)

    return call(pack32(h), s, r, o, pack32(W))
